# Initial kernel scaffold; baseline (speedup 1.0000x reference)
import jax
import jax.numpy as jnp
from jax.experimental import pallas as pl


def kernel(x, mask, w1, g1, b1, rm1, rv1, w2, g2, b2, rm2, rv2, w3, g3, b3, rm3, rv3, inference):
    raise NotImplementedError("write your pallas kernel here")


# speedup vs baseline: 2.0579x; 2.0579x over previous
import jax
import jax.numpy as jnp
from jax.experimental import pallas as pl
from jax.experimental.pallas import tpu as pltpu

TH = 64
HB = 8


def _body(xm_ref, xu_ref, xd_ref, mm_ref, mu_ref, md_ref,
          w1_ref, b1_ref, w2_ref, b2_ref, w3_ref, b3_ref, o_ref):
    t = pl.program_id(1)
    nt = pl.num_programs(1)
    C, Th, W = xm_ref.shape[1], xm_ref.shape[2], xm_ref.shape[3]
    Cm = w1_ref.shape[0]
    Rh = Th + 2

    xm = xm_ref[0]
    x_ext = jnp.concatenate(
        [xu_ref[0, :, HB - 1:HB], xm, xd_ref[0, :, 0:1]], axis=1)
    m_ext = jnp.concatenate(
        [mu_ref[0, HB - 1:HB], mm_ref[0], md_ref[0, 0:1]], axis=0)

    a = (x_ext * m_ext[None]).reshape(C, Rh * W)
    t1 = jnp.dot(w1_ref[...], a, preferred_element_type=jnp.float32)
    t1 = jnp.maximum(t1 + b1_ref[...], 0.0).reshape(Cm, Rh, W) * m_ext[None]

    rows = jax.lax.broadcasted_iota(jnp.int32, (1, Rh, 1), 1)
    bad = ((rows == 0) & (t == 0)) | ((rows == Rh - 1) & (t == nt - 1))
    t1 = jnp.where(bad, 0.0, t1)

    zcol = jnp.zeros((Cm, Rh, 1), jnp.float32)
    t1p = jnp.concatenate([zcol, t1, zcol], axis=2)

    acc = jnp.zeros((Cm, Th * W), jnp.float32)
    for k in range(9):
        dy, dx = k // 3, k % 3
        sl = t1p[:, dy:dy + Th, dx:dx + W].reshape(Cm, Th * W)
        acc = acc + jnp.dot(w2_ref[k], sl, preferred_element_type=jnp.float32)
    t2 = jnp.maximum(acc + b2_ref[...], 0.0).reshape(Cm, Th, W) * mm_ref[0][None]

    out = jnp.dot(w3_ref[...], t2.reshape(Cm, Th * W),
                  preferred_element_type=jnp.float32) + b3_ref[...]
    o_ref[0] = jnp.maximum(out.reshape(C, Th, W) + xm, 0.0)


def kernel(x, mask, w1, g1, b1, rm1, rv1, w2, g2, b2, rm2, rv2,
           w3, g3, b3, rm3, rv3, inference=False):
    B, C, H, W = x.shape
    Cm = w1.shape[0]
    mh, mw = mask.shape[2], mask.shape[3]

    s1 = g1 / jnp.sqrt(rv1 + 1e-5)
    s2 = g2 / jnp.sqrt(rv2 + 1e-5)
    s3 = g3 / jnp.sqrt(rv3 + 1e-5)
    w1f = w1[:, :, 0, 0] * s1[:, None]
    b1f = (b1 - rm1 * s1)[:, None]
    w2f = jnp.transpose(w2 * s2[:, None, None, None], (2, 3, 0, 1)).reshape(9, Cm, Cm)
    b2f = (b2 - rm2 * s2)[:, None]
    w3f = w3[:, :, 0, 0] * s3[:, None]
    b3f = (b3 - rm3 * s3)[:, None]

    mfull = jnp.broadcast_to(mask[:, 0, :, None, :, None],
                             (B, mh, H // mh, mw, W // mw)).reshape(B, H, W)

    nt = H // TH
    nhb = H // HB
    rb = TH // HB
    grid = (B, nt)

    out = pl.pallas_call(
        _body,
        grid=grid,
        in_specs=[
            pl.BlockSpec((1, C, TH, W), lambda b, t: (b, 0, t, 0)),
            pl.BlockSpec((1, C, HB, W), lambda b, t: (b, 0, jnp.maximum(t * rb - 1, 0), 0)),
            pl.BlockSpec((1, C, HB, W), lambda b, t: (b, 0, jnp.minimum(t * rb + rb, nhb - 1), 0)),
            pl.BlockSpec((1, TH, W), lambda b, t: (b, t, 0)),
            pl.BlockSpec((1, HB, W), lambda b, t: (b, jnp.maximum(t * rb - 1, 0), 0)),
            pl.BlockSpec((1, HB, W), lambda b, t: (b, jnp.minimum(t * rb + rb, nhb - 1), 0)),
            pl.BlockSpec((Cm, C), lambda b, t: (0, 0)),
            pl.BlockSpec((Cm, 1), lambda b, t: (0, 0)),
            pl.BlockSpec((9, Cm, Cm), lambda b, t: (0, 0, 0)),
            pl.BlockSpec((Cm, 1), lambda b, t: (0, 0)),
            pl.BlockSpec((C, Cm), lambda b, t: (0, 0)),
            pl.BlockSpec((C, 1), lambda b, t: (0, 0)),
        ],
        out_specs=pl.BlockSpec((1, C, TH, W), lambda b, t: (b, 0, t, 0)),
        out_shape=jax.ShapeDtypeStruct((B, C, H, W), jnp.float32),
        compiler_params=pltpu.CompilerParams(
            dimension_semantics=("parallel", "arbitrary")),
    )(x, x, x, mfull, mfull, mfull, w1f, b1f, w2f, b2f, w3f, b3f)
    return out
